# baseline (speedup 1.0000x reference)
import functools

import jax
import jax.numpy as jnp
from jax import lax
from jax.experimental import pallas as pl
from jax.experimental.pallas import tpu as pltpu
from jax.experimental.pallas import tpu_sc as plsc

VOCAB = 100000
EMB = 128
BATCH = 4096
MAXLEN = 200

_VB = 5
_VROWS = VOCAB // _VB

_NC = 2
_NS = 16
_NW = _NC * _NS
_RPW = BATCH // _NW
_IPW = _RPW * MAXLEN
_LANES = 16


def _fold_body(x_ref, w_ref, b_ref, o_ref):
    x = x_ref[...]
    w = w_ref[...]
    s = jax.lax.dot_general(
        w, x, (((1,), (1,)), ((), ())),
        preferred_element_type=jnp.float32)
    o_ref[0, 0, :] = (s[0] + b_ref[0, 0]) * (1.0 / MAXLEN)


def _fold_table(x2, w2, b2):
    return pl.pallas_call(
        _fold_body,
        grid=(_VB,),
        in_specs=[
            pl.BlockSpec((_VROWS, EMB), lambda i: (i, 0)),
            pl.BlockSpec((1, EMB), lambda i: (0, 0)),
            pl.BlockSpec((1, 1), lambda i: (0, 0)),
        ],
        out_specs=pl.BlockSpec((1, 1, _VROWS), lambda i: (i, 0, 0)),
        out_shape=jax.ShapeDtypeStruct((_VB, 1, _VROWS), jnp.float32),
    )(x2, w2, b2)


def _sc_pool(t_flat, idx2d):
    mesh = plsc.VectorSubcoreMesh(core_axis_name="c", subcore_axis_name="s")

    @functools.partial(
        pl.kernel,
        mesh=mesh,
        out_type=jax.ShapeDtypeStruct((BATCH,), jnp.float32),
        scratch_types=[
            pltpu.VMEM((VOCAB,), jnp.float32),
            pltpu.VMEM_SHARED((VOCAB,), jnp.float32),
            pltpu.VMEM((_RPW // 2, MAXLEN), jnp.int32),
            pltpu.VMEM((_RPW,), jnp.float32),
            pltpu.SemaphoreType.DMA,
            pltpu.SemaphoreType.DMA,
        ],
        compiler_params=pltpu.CompilerParams(needs_layout_passes=False),
    )
    def k(t_hbm, idx_hbm, out_hbm, t_v, t_sh, idx_v, out_v, sem, sem2):
        sid = lax.axis_index("s")
        wid = sid * _NC + lax.axis_index("c")
        base = wid * _RPW
        half = _RPW // 2
        shard = _VROWS
        idx_cp = pltpu.async_copy(
            idx_hbm.at[pl.ds(base, half), :], idx_v, sem2)
        for r in range((_VB + _NS - 1) // _NS):
            c = r * _NS + sid

            @pl.when(c < _VB)
            def _(c=c):
                pltpu.sync_copy(t_hbm.at[c, 0, :],
                                t_v.at[pl.ds(c * shard, shard)])
                pltpu.sync_copy(t_v.at[pl.ds(c * shard, shard)],
                                t_sh.at[pl.ds(c * shard, shard)])
        plsc.subcore_barrier()
        t_cp = pltpu.async_copy(t_sh, t_v, sem)
        idx_cp.wait()
        t_cp.wait()

        lane = lax.iota(jnp.int32, _LANES)

        for h in range(2):
            if h:
                pltpu.sync_copy(idx_hbm.at[pl.ds(base + half, half), :], idx_v)

            def g_body(g, carry):
                rows = lane + g * _LANES
                acc = jnp.zeros((_LANES,), jnp.float32)
                for l in range(MAXLEN):
                    cols = jnp.full((_LANES,), l, jnp.int32)
                    ivals = plsc.load_gather(idx_v, [rows, cols])
                    acc = acc + plsc.load_gather(t_v, [ivals])
                out_v[pl.ds(h * half + g * _LANES, _LANES)] = (
                    1.0 / (1.0 + jnp.exp(-acc)))
                return carry

            lax.fori_loop(0, half // _LANES, g_body, 0)
        pltpu.sync_copy(out_v, out_hbm.at[pl.ds(base, _RPW)])

    return k(t_flat, idx2d)


def kernel(indices, emb_table, W, b):
    w2 = W.reshape(1, EMB)
    b2 = b.reshape(1, 1)
    t = _fold_table(emb_table, w2, b2)
    out = _sc_pool(t, indices.astype(jnp.int32))
    return out.reshape(BATCH, 1)


# speedup vs baseline: 1.0795x; 1.0795x over previous
import functools

import jax
import jax.numpy as jnp
from jax import lax
from jax.experimental import pallas as pl
from jax.experimental.pallas import tpu as pltpu
from jax.experimental.pallas import tpu_sc as plsc

VOCAB = 100000
EMB = 128
BATCH = 4096
MAXLEN = 200

_VB = 5
_VROWS = VOCAB // _VB

_NC = 2
_NS = 16
_NW = _NC * _NS
_RPW = BATCH // _NW
_IPW = _RPW * MAXLEN
_LANES = 16


def _fold_body(x_ref, w_ref, b_ref, o_ref):
    x = x_ref[...]
    w = w_ref[...]
    s = jax.lax.dot_general(
        w, x, (((1,), (1,)), ((), ())),
        preferred_element_type=jnp.float32)
    o_ref[0, 0, :] = (s[0] + b_ref[0, 0]) * (1.0 / MAXLEN)


def _fold_table(x2, w2, b2):
    return pl.pallas_call(
        _fold_body,
        grid=(_VB,),
        in_specs=[
            pl.BlockSpec((_VROWS, EMB), lambda i: (i, 0)),
            pl.BlockSpec((1, EMB), lambda i: (0, 0)),
            pl.BlockSpec((1, 1), lambda i: (0, 0)),
        ],
        out_specs=pl.BlockSpec((1, 1, _VROWS), lambda i: (i, 0, 0)),
        out_shape=jax.ShapeDtypeStruct((_VB, 1, _VROWS), jnp.float32),
    )(x2, w2, b2)


def _sc_pool(t_flat, idx2d):
    mesh = plsc.VectorSubcoreMesh(core_axis_name="c", subcore_axis_name="s")

    @functools.partial(
        pl.kernel,
        mesh=mesh,
        out_type=jax.ShapeDtypeStruct((BATCH,), jnp.float32),
        scratch_types=[
            pltpu.VMEM((VOCAB,), jnp.float32),
            pltpu.VMEM_SHARED((VOCAB,), jnp.float32),
            pltpu.VMEM((_RPW // 2, MAXLEN), jnp.int32),
            pltpu.VMEM((_RPW,), jnp.float32),
            pltpu.SemaphoreType.DMA,
            pltpu.SemaphoreType.DMA,
        ],
        compiler_params=pltpu.CompilerParams(needs_layout_passes=False),
    )
    def k(t_hbm, idx_hbm, out_hbm, t_v, t_sh, idx_v, out_v, sem, sem2):
        sid = lax.axis_index("s")
        wid = sid * _NC + lax.axis_index("c")
        base = wid * _RPW
        half = _RPW // 2
        shard = _VROWS
        idx_cp = pltpu.async_copy(
            idx_hbm.at[pl.ds(base, half), :], idx_v, sem2)
        for r in range((_VB + _NS - 1) // _NS):
            c = r * _NS + sid

            @pl.when(c < _VB)
            def _(c=c):
                pltpu.sync_copy(t_hbm.at[c, 0, :],
                                t_v.at[pl.ds(c * shard, shard)])
                pltpu.sync_copy(t_v.at[pl.ds(c * shard, shard)],
                                t_sh.at[pl.ds(c * shard, shard)])
        plsc.subcore_barrier()
        t_cp = pltpu.async_copy(t_sh, t_v, sem)
        idx_cp.wait()
        t_cp.wait()

        lane = lax.iota(jnp.int32, _LANES)

        for h in range(2):
            if h:
                pltpu.sync_copy(idx_hbm.at[pl.ds(base + half, half), :], idx_v)

            def g_body(g, carry):
                rows = lane + g * _LANES

                def lc_body(lc, acc):
                    lbase = lc * 25
                    for j in range(25):
                        cols = jnp.full((_LANES,), lbase + j, jnp.int32)
                        ivals = plsc.load_gather(idx_v, [rows, cols])
                        acc = acc + plsc.load_gather(t_v, [ivals])
                    return acc

                acc = lax.fori_loop(0, MAXLEN // 25, lc_body,
                                    jnp.zeros((_LANES,), jnp.float32))
                out_v[pl.ds(h * half + g * _LANES, _LANES)] = (
                    1.0 / (1.0 + jnp.exp(-acc)))
                return carry

            lax.fori_loop(0, half // _LANES, g_body, 0)
        pltpu.sync_copy(out_v, out_hbm.at[pl.ds(base, _RPW)])

    return k(t_flat, idx2d)


def kernel(indices, emb_table, W, b):
    w2 = W.reshape(1, EMB)
    b2 = b.reshape(1, 1)
    t = _fold_table(emb_table, w2, b2)
    out = _sc_pool(t, indices.astype(jnp.int32))
    return out.reshape(BATCH, 1)
